# baseline (speedup 1.0000x reference)
import functools

import jax
import jax.numpy as jnp
from jax import lax
from jax.experimental import pallas as pl
from jax.experimental.pallas import tpu as pltpu
from jax.experimental.pallas import tpu_sc as plsc

N = 10000
D = 128
E = 320000

NC = 2
NS = 16
NW = NC * NS
EPT = E // NW
K = 40
C = EPT // K
SEG = 5
CS = C // SEG
NB = 6
KD = 80
CD = EPT // KD
IO_T = 10
RPT = N // IO_T
ZR = 40

_mesh = plsc.VectorSubcoreMesh(core_axis_name="c", subcore_axis_name="s")


@functools.partial(
    pl.kernel,
    out_type=jax.ShapeDtypeStruct((NC, N, D), jnp.float32),
    mesh=_mesh,
    scratch_types=[
        pltpu.VMEM((CS, K), jnp.int32),
        pltpu.VMEM((CS, K), jnp.int32),
        [pltpu.VMEM((K, D), jnp.float32)] * NB,
        [pltpu.SemaphoreType.DMA] * NB,
        [pltpu.SemaphoreType.DMA] * NB,
        pltpu.VMEM_SHARED((N, D), jnp.float32),
    ],
)
def _sc_aggregate(h_hbm, src_hbm, dst_hbm, zeros_hbm, out_hbm,
                  sidx, didx, rows, gsem, ssem, acc):
    cid = lax.axis_index("c")
    sid = lax.axis_index("s")
    w = cid * NS + sid

    def gather(c, b):
        return pltpu.async_copy(h_hbm.at[sidx.at[c]], rows[b], gsem[b])

    def gather_wait(c, b):
        pltpu.make_async_copy(h_hbm.at[sidx.at[c]], rows[b], gsem[b]).wait()

    def scat(c, b):
        return pltpu.async_copy(rows[b], acc.at[didx.at[c]], ssem[b],
                                add=True)

    def scat_wait(c, b):
        pltpu.make_async_copy(rows[b], acc.at[didx.at[c]], ssem[b]).wait()

    @pl.when(jnp.logical_and(sid < IO_T, cid == 0))
    def _():
        nj = RPT // ZR
        for j in range(nj):
            b, sm = rows[j % 2].at[pl.ds(0, ZR)], ssem[j % 2]
            if j >= 2:
                pltpu.make_async_copy(
                    b, acc.at[pl.ds(sid * RPT + (j - 2) * ZR, ZR)], sm).wait()
            pltpu.sync_copy(h_hbm.at[pl.ds(sid * RPT + j * ZR, ZR)], b)
            pltpu.async_copy(b, acc.at[pl.ds(sid * RPT + j * ZR, ZR)], sm)
        for j in (nj - 2, nj - 1):
            pltpu.make_async_copy(
                rows[j % 2].at[pl.ds(0, ZR)],
                acc.at[pl.ds(sid * RPT + j * ZR, ZR)], ssem[j % 2]).wait()

    @pl.when(jnp.logical_and(sid < IO_T, cid == 1))
    def _():
        z = rows[0].at[pl.ds(0, ZR)]
        pltpu.sync_copy(zeros_hbm, z)
        for j in range(RPT // ZR):
            pltpu.async_copy(z, acc.at[pl.ds(sid * RPT + j * ZR, ZR)],
                             ssem[0])
        for j in range(RPT // ZR):
            pltpu.make_async_copy(
                z, acc.at[pl.ds(sid * RPT + j * ZR, ZR)], ssem[0]).wait()

    plsc.subcore_barrier()

    def ring_round(j, carry):
        base = NB * j
        for b in range(NB):
            gather_wait(base + b, b)
            scat(base + b, b)
        for b in range(NB):
            scat_wait(base + b, b)
            gather(base + NB + b, b)
        return carry

    for s in range(SEG):
        pltpu.sync_copy(src_hbm.at[w, s], sidx)
        pltpu.sync_copy(dst_hbm.at[w, s], didx)
        for b in range(NB):
            gather(b, b)
        J = CS // NB - 1
        lax.fori_loop(0, J, ring_round, 0)
        baseT = NB * J
        R = CS - baseT - NB
        for b in range(NB):
            gather_wait(baseT + b, b)
            scat(baseT + b, b)
        for b in range(R):
            scat_wait(baseT + b, b)
            gather(baseT + NB + b, b)
        for b in range(R):
            gather_wait(baseT + NB + b, b)
            scat(baseT + NB + b, b)
        for b in range(R, NB):
            scat_wait(baseT + b, b)
        for b in range(R):
            scat_wait(baseT + NB + b, b)
    plsc.subcore_barrier()

    @pl.when(sid < IO_T)
    def _():
        nj = RPT // ZR
        for j in range(nj):
            b, sm = rows[j % 2].at[pl.ds(0, ZR)], ssem[j % 2]
            if j >= 2:
                pltpu.make_async_copy(
                    b, out_hbm.at[cid, pl.ds(sid * RPT + (j - 2) * ZR, ZR)],
                    sm).wait()
            pltpu.sync_copy(acc.at[pl.ds(sid * RPT + j * ZR, ZR)], b)
            pltpu.async_copy(
                b, out_hbm.at[cid, pl.ds(sid * RPT + j * ZR, ZR)], sm)
        for j in (nj - 2, nj - 1):
            pltpu.make_async_copy(
                rows[j % 2].at[pl.ds(0, ZR)],
                out_hbm.at[cid, pl.ds(sid * RPT + j * ZR, ZR)],
                ssem[j % 2]).wait()


@functools.partial(
    pl.kernel,
    out_type=jax.ShapeDtypeStruct((NC * N,), jnp.float32),
    mesh=_mesh,
    scratch_types=[
        pltpu.VMEM((CD, KD), jnp.int32),
        pltpu.VMEM((KD,), jnp.float32),
        pltpu.VMEM((RPT,), jnp.float32),
        pltpu.VMEM_SHARED((N,), jnp.float32),
        pltpu.SemaphoreType.DMA,
    ],
)
def _sc_degree(dst_hbm, zeros_hbm, out_hbm, didx, ones_v, dbuf, acc, sem):
    cid = lax.axis_index("c")
    sid = lax.axis_index("s")
    w = cid * NS + sid

    @pl.when(sid < IO_T)
    def _():
        pltpu.sync_copy(zeros_hbm, dbuf)
        pltpu.sync_copy(dbuf, acc.at[pl.ds(sid * RPT, RPT)])

    pltpu.sync_copy(dst_hbm.at[w], didx)
    for i in range(KD // 16):
        ones_v[pl.ds(i * 16, 16)] = jnp.ones((16,), jnp.float32)
    plsc.subcore_barrier()

    def chunk(c, carry):
        pltpu.async_copy(ones_v, acc.at[didx.at[c]], sem, add=True)
        return carry

    lax.fori_loop(0, CD, chunk, 0)

    def drain(c, carry):
        pltpu.make_async_copy(ones_v, acc.at[didx.at[c]], sem).wait()
        return carry

    lax.fori_loop(0, CD, drain, 0)
    plsc.subcore_barrier()

    @pl.when(sid < IO_T)
    def _():
        pltpu.sync_copy(acc.at[pl.ds(sid * RPT, RPT)], dbuf)
        pltpu.sync_copy(dbuf, out_hbm.at[pl.ds(cid * N + sid * RPT, RPT)])


_BM = 2000


def _dinv_block(dref):
    d = dref[...]
    deg = d[:, 0:1] + d[:, 1:2] + 1.0
    return lax.rsqrt(jnp.maximum(deg, 1.0))


def _prep_body(xref, wref, dref, out):
    out[...] = jnp.dot(xref[...], wref[...],
                       preferred_element_type=jnp.float32) * _dinv_block(dref)


def _layer_body(aref, wref, bref, dref, out):
    dinv = _dinv_block(dref)
    p = aref[0] + aref[1]
    t = jnp.maximum(p * dinv + bref[...], 0.0)
    out[...] = jnp.dot(t, wref[...],
                       preferred_element_type=jnp.float32) * dinv


def _final_body(aref, bref, dref, out):
    dinv = _dinv_block(dref)
    out[...] = (aref[0] + aref[1]) * dinv + bref[...]


_row_spec = pl.BlockSpec((_BM, D), lambda i: (i, 0))
_agg_spec = pl.BlockSpec((NC, _BM, D), lambda i: (0, i, 0))
_w_spec = pl.BlockSpec((D, D), lambda i: (0, 0))
_b_spec = pl.BlockSpec((1, D), lambda i: (0, 0))
_deg_spec = pl.BlockSpec((_BM, NC), lambda i: (i, 0))
_out_f32 = jax.ShapeDtypeStruct((N, D), jnp.float32)

_prep = pl.pallas_call(
    _prep_body, grid=(N // _BM,),
    in_specs=[_row_spec, _w_spec, _deg_spec],
    out_specs=_row_spec, out_shape=_out_f32)

_layer = pl.pallas_call(
    _layer_body, grid=(N // _BM,),
    in_specs=[_agg_spec, _w_spec, _b_spec, _deg_spec],
    out_specs=_row_spec, out_shape=_out_f32)

_final = pl.pallas_call(
    _final_body, grid=(N // _BM,),
    in_specs=[_agg_spec, _b_spec, _deg_spec],
    out_specs=_row_spec, out_shape=_out_f32)


def kernel(x, edge_index, W1, b1, W2, b2, W3, b3):
    src = edge_index[0].reshape(NW, SEG, CS, K)
    dst = edge_index[1].reshape(NW, SEG, CS, K)
    dst_flat = edge_index[1].reshape(NW, CD, KD)
    zeros_rows = jnp.zeros((ZR, D), jnp.float32)
    zeros_deg = jnp.zeros((RPT,), jnp.float32)
    b1r = b1.reshape(1, D)
    b2r = b2.reshape(1, D)
    b3r = b3.reshape(1, D)

    degp = _sc_degree(dst_flat, zeros_deg).reshape(NC, N).T
    h1 = _prep(x, W1, degp)
    a1 = _sc_aggregate(h1, src, dst, zeros_rows)
    h2 = _layer(a1, W2, b1r, degp)
    a2 = _sc_aggregate(h2, src, dst, zeros_rows)
    h3 = _layer(a2, W3, b2r, degp)
    a3 = _sc_aggregate(h3, src, dst, zeros_rows)
    return _final(a3, b3r, degp)


# speedup vs baseline: 1.0924x; 1.0924x over previous
import functools

import jax
import jax.numpy as jnp
from jax import lax
from jax.experimental import pallas as pl
from jax.experimental.pallas import tpu as pltpu
from jax.experimental.pallas import tpu_sc as plsc

N = 10000
D = 128
E = 320000

NC = 2
NS = 16
NW = NC * NS
EPT = E // NW
K = 40
C = EPT // K
SEG = 5
CS = C // SEG
NB = 6
KD = 80
CD = EPT // KD
IO_T = 10
RPT = N // IO_T
ZR = 40

_mesh = plsc.VectorSubcoreMesh(core_axis_name="c", subcore_axis_name="s")


@functools.partial(
    pl.kernel,
    out_type=jax.ShapeDtypeStruct((NC, N, D), jnp.float32),
    mesh=_mesh,
    scratch_types=[
        pltpu.VMEM((CS, K), jnp.int32),
        pltpu.VMEM((CS, K), jnp.int32),
        [pltpu.VMEM((K, D), jnp.float32)] * NB,
        [pltpu.SemaphoreType.DMA] * NB,
        [pltpu.SemaphoreType.DMA] * NB,
        pltpu.VMEM_SHARED((N, D), jnp.float32),
    ],
)
def _sc_aggregate(h_hbm, src_hbm, dst_hbm, zeros_hbm, out_hbm,
                  sidx, didx, rows, gsem, ssem, acc):
    cid = lax.axis_index("c")
    sid = lax.axis_index("s")
    w = cid * NS + sid

    def gather(c, b):
        return pltpu.async_copy(h_hbm.at[sidx.at[c]], rows[b], gsem[b])

    def gather_wait(c, b):
        pltpu.make_async_copy(h_hbm.at[sidx.at[c]], rows[b], gsem[b]).wait()

    def scat(c, b):
        return pltpu.async_copy(rows[b], acc.at[didx.at[c]], ssem[b],
                                add=True)

    def scat_wait(c, b):
        pltpu.make_async_copy(rows[b], acc.at[didx.at[c]], ssem[b]).wait()

    @pl.when(sid < IO_T)
    def _():
        z = rows[0].at[pl.ds(0, ZR)]
        pltpu.sync_copy(zeros_hbm, z)
        for j in range(RPT // ZR):
            pltpu.async_copy(z, acc.at[pl.ds(sid * RPT + j * ZR, ZR)],
                             ssem[0])
        for j in range(RPT // ZR):
            pltpu.make_async_copy(
                z, acc.at[pl.ds(sid * RPT + j * ZR, ZR)], ssem[0]).wait()

    plsc.subcore_barrier()

    def ring_round(j, carry):
        base = NB * j
        for b in range(NB):
            gather_wait(base + b, b)
            scat(base + b, b)
        for b in range(NB):
            scat_wait(base + b, b)
            gather(base + NB + b, b)
        return carry

    for s in range(SEG):
        pltpu.sync_copy(src_hbm.at[w, s], sidx)
        pltpu.sync_copy(dst_hbm.at[w, s], didx)
        for b in range(NB):
            gather(b, b)
        J = CS // NB - 1
        lax.fori_loop(0, J, ring_round, 0)
        baseT = NB * J
        R = CS - baseT - NB
        for b in range(NB):
            gather_wait(baseT + b, b)
            scat(baseT + b, b)
        for b in range(R):
            scat_wait(baseT + b, b)
            gather(baseT + NB + b, b)
        for b in range(R):
            gather_wait(baseT + NB + b, b)
            scat(baseT + NB + b, b)
        for b in range(R, NB):
            scat_wait(baseT + b, b)
        for b in range(R):
            scat_wait(baseT + NB + b, b)
    plsc.subcore_barrier()

    @pl.when(sid < IO_T)
    def _():
        nj = RPT // ZR
        for j in range(nj):
            b, sm = rows[j % 2].at[pl.ds(0, ZR)], ssem[j % 2]
            if j >= 2:
                pltpu.make_async_copy(
                    b, out_hbm.at[cid, pl.ds(sid * RPT + (j - 2) * ZR, ZR)],
                    sm).wait()
            pltpu.sync_copy(acc.at[pl.ds(sid * RPT + j * ZR, ZR)], b)
            pltpu.async_copy(
                b, out_hbm.at[cid, pl.ds(sid * RPT + j * ZR, ZR)], sm)
        for j in (nj - 2, nj - 1):
            pltpu.make_async_copy(
                rows[j % 2].at[pl.ds(0, ZR)],
                out_hbm.at[cid, pl.ds(sid * RPT + j * ZR, ZR)],
                ssem[j % 2]).wait()


@functools.partial(
    pl.kernel,
    out_type=jax.ShapeDtypeStruct((NC * N,), jnp.float32),
    mesh=_mesh,
    scratch_types=[
        pltpu.VMEM((CD, KD), jnp.int32),
        pltpu.VMEM((KD,), jnp.float32),
        pltpu.VMEM((RPT,), jnp.float32),
        pltpu.VMEM_SHARED((N,), jnp.float32),
        pltpu.SemaphoreType.DMA,
    ],
)
def _sc_degree(dst_hbm, zeros_hbm, out_hbm, didx, ones_v, dbuf, acc, sem):
    cid = lax.axis_index("c")
    sid = lax.axis_index("s")
    w = cid * NS + sid

    @pl.when(sid < IO_T)
    def _():
        pltpu.sync_copy(zeros_hbm, dbuf)
        pltpu.sync_copy(dbuf, acc.at[pl.ds(sid * RPT, RPT)])

    pltpu.sync_copy(dst_hbm.at[w], didx)
    for i in range(KD // 16):
        ones_v[pl.ds(i * 16, 16)] = jnp.ones((16,), jnp.float32)
    plsc.subcore_barrier()

    def chunk(c, carry):
        pltpu.async_copy(ones_v, acc.at[didx.at[c]], sem, add=True)
        return carry

    lax.fori_loop(0, CD, chunk, 0)

    def drain(c, carry):
        pltpu.make_async_copy(ones_v, acc.at[didx.at[c]], sem).wait()
        return carry

    lax.fori_loop(0, CD, drain, 0)
    plsc.subcore_barrier()

    @pl.when(sid < IO_T)
    def _():
        pltpu.sync_copy(acc.at[pl.ds(sid * RPT, RPT)], dbuf)
        pltpu.sync_copy(dbuf, out_hbm.at[pl.ds(cid * N + sid * RPT, RPT)])


_BM = 2000


def _dinv_block(dref):
    d = dref[...]
    deg = d[:, 0:1] + d[:, 1:2] + 1.0
    return lax.rsqrt(jnp.maximum(deg, 1.0))


def _prep_body(xref, wref, dref, out):
    out[...] = jnp.dot(xref[...], wref[...],
                       preferred_element_type=jnp.float32) * _dinv_block(dref)


def _layer_body(aref, href, wref, bref, dref, out):
    dinv = _dinv_block(dref)
    p = aref[0] + aref[1] + href[...]
    t = jnp.maximum(p * dinv + bref[...], 0.0)
    out[...] = jnp.dot(t, wref[...],
                       preferred_element_type=jnp.float32) * dinv


def _final_body(aref, href, bref, dref, out):
    dinv = _dinv_block(dref)
    out[...] = (aref[0] + aref[1] + href[...]) * dinv + bref[...]


_row_spec = pl.BlockSpec((_BM, D), lambda i: (i, 0))
_agg_spec = pl.BlockSpec((NC, _BM, D), lambda i: (0, i, 0))
_w_spec = pl.BlockSpec((D, D), lambda i: (0, 0))
_b_spec = pl.BlockSpec((1, D), lambda i: (0, 0))
_deg_spec = pl.BlockSpec((_BM, NC), lambda i: (i, 0))
_out_f32 = jax.ShapeDtypeStruct((N, D), jnp.float32)

_prep = pl.pallas_call(
    _prep_body, grid=(N // _BM,),
    in_specs=[_row_spec, _w_spec, _deg_spec],
    out_specs=_row_spec, out_shape=_out_f32)

_layer = pl.pallas_call(
    _layer_body, grid=(N // _BM,),
    in_specs=[_agg_spec, _row_spec, _w_spec, _b_spec, _deg_spec],
    out_specs=_row_spec, out_shape=_out_f32)

_final = pl.pallas_call(
    _final_body, grid=(N // _BM,),
    in_specs=[_agg_spec, _row_spec, _b_spec, _deg_spec],
    out_specs=_row_spec, out_shape=_out_f32)


def kernel(x, edge_index, W1, b1, W2, b2, W3, b3):
    src = edge_index[0].reshape(NW, SEG, CS, K)
    dst = edge_index[1].reshape(NW, SEG, CS, K)
    dst_flat = edge_index[1].reshape(NW, CD, KD)
    zeros_rows = jnp.zeros((ZR, D), jnp.float32)
    zeros_deg = jnp.zeros((RPT,), jnp.float32)
    b1r = b1.reshape(1, D)
    b2r = b2.reshape(1, D)
    b3r = b3.reshape(1, D)

    degp = _sc_degree(dst_flat, zeros_deg).reshape(NC, N).T
    h1 = _prep(x, W1, degp)
    a1 = _sc_aggregate(h1, src, dst, zeros_rows)
    h2 = _layer(a1, h1, W2, b1r, degp)
    a2 = _sc_aggregate(h2, src, dst, zeros_rows)
    h3 = _layer(a2, h2, W3, b2r, degp)
    a3 = _sc_aggregate(h3, src, dst, zeros_rows)
    return _final(a3, h3, b3r, degp)


# speedup vs baseline: 1.1478x; 1.0506x over previous
import functools

import jax
import jax.numpy as jnp
from jax import lax
from jax.experimental import pallas as pl
from jax.experimental.pallas import tpu as pltpu
from jax.experimental.pallas import tpu_sc as plsc

N = 10000
D = 128
E = 320000

NC = 2
NS = 16
NW = NC * NS
EPT = E // NW
K = 40
C = EPT // K
SEG = 10
CS = C // SEG
NB = 5
NJ = N // 40
KD = 80
CD = EPT // KD
IO_T = 10
RPT = N // IO_T
ZR = 40

_mesh = plsc.VectorSubcoreMesh(core_axis_name="c", subcore_axis_name="s")


@functools.partial(
    pl.kernel,
    out_type=jax.ShapeDtypeStruct((NC, N, D), jnp.float32),
    mesh=_mesh,
    scratch_types=[
        [pltpu.VMEM((CS, K), jnp.int32)] * 2,
        [pltpu.VMEM((CS, K), jnp.int32)] * 2,
        [pltpu.VMEM((K, D), jnp.float32)] * NB,
        [pltpu.SemaphoreType.DMA] * NB,
        [pltpu.SemaphoreType.DMA] * NB,
        pltpu.SemaphoreType.DMA,
        pltpu.VMEM_SHARED((N, D), jnp.float32),
    ],
)
def _sc_aggregate(h_hbm, src_hbm, dst_hbm, zeros_hbm, out_hbm,
                  sidxs, didxs, rows, gsem, ssem, isem, acc):
    cid = lax.axis_index("c")
    sid = lax.axis_index("s")
    w = cid * NS + sid

    def gather(si, c, b):
        return pltpu.async_copy(h_hbm.at[si.at[c]], rows[b], gsem[b])

    def gather_wait(si, c, b):
        pltpu.make_async_copy(h_hbm.at[si.at[c]], rows[b], gsem[b]).wait()

    def scat(di, c, b):
        return pltpu.async_copy(rows[b], acc.at[di.at[c]], ssem[b], add=True)

    def scat_wait(di, c, b):
        pltpu.make_async_copy(rows[b], acc.at[di.at[c]], ssem[b]).wait()

    z = rows[0].at[pl.ds(0, ZR)]
    pltpu.sync_copy(zeros_hbm, z)
    for r in range(NJ // NS + 1):
        j = r * NS + sid

        @pl.when(j < NJ)
        def _():
            pltpu.async_copy(z, acc.at[pl.ds(j * ZR, ZR)], ssem[0])
    for r in range(NJ // NS + 1):
        j = r * NS + sid

        @pl.when(j < NJ)
        def _():
            pltpu.make_async_copy(z, acc.at[pl.ds(j * ZR, ZR)],
                                  ssem[0]).wait()

    pltpu.sync_copy(src_hbm.at[w, 0], sidxs[0])
    pltpu.sync_copy(dst_hbm.at[w, 0], didxs[0])
    plsc.subcore_barrier()
    for b in range(NB):
        gather(sidxs[0], b, b)

    for s in range(SEG):
        si, di = sidxs[s % 2], didxs[s % 2]
        sn, dn = sidxs[(s + 1) % 2], didxs[(s + 1) % 2]
        if s + 1 < SEG:
            pltpu.async_copy(src_hbm.at[w, s + 1], sn, isem)
            pltpu.async_copy(dst_hbm.at[w, s + 1], dn, isem)

        def ring_round(j, carry, si=si, di=di):
            base = NB * j
            for b in range(NB):
                gather_wait(si, base + b, b)
                scat(di, base + b, b)
            for b in range(NB):
                scat_wait(di, base + b, b)
                gather(si, base + NB + b, b)
            return carry

        lax.fori_loop(0, CS // NB - 1, ring_round, 0)
        base = CS - NB
        if s + 1 < SEG:
            pltpu.make_async_copy(src_hbm.at[w, s + 1], sn, isem).wait()
            pltpu.make_async_copy(dst_hbm.at[w, s + 1], dn, isem).wait()
        for b in range(NB):
            gather_wait(si, base + b, b)
            scat(di, base + b, b)
        for b in range(NB):
            scat_wait(di, base + b, b)
            if s + 1 < SEG:
                gather(sn, b, b)
    plsc.subcore_barrier()

    for r in range(NJ // NS + 1):
        j = r * NS + sid

        @pl.when(j < NJ)
        def _():
            b, sm = rows[r % 2].at[pl.ds(0, ZR)], ssem[r % 2]
            if r >= 2:
                jp = (r - 2) * NS + sid
                pltpu.make_async_copy(
                    b, out_hbm.at[cid, pl.ds(jp * ZR, ZR)], sm).wait()
            pltpu.sync_copy(acc.at[pl.ds(j * ZR, ZR)], b)
            pltpu.async_copy(b, out_hbm.at[cid, pl.ds(j * ZR, ZR)], sm)

    _r_last = NJ // NS
    @pl.when(sid < NJ % NS)
    def _():
        j = _r_last * NS + sid
        pltpu.make_async_copy(
            rows[_r_last % 2].at[pl.ds(0, ZR)],
            out_hbm.at[cid, pl.ds(j * ZR, ZR)], ssem[_r_last % 2]).wait()

    @pl.when(sid >= NJ % NS)
    def _():
        j = (_r_last - 2) * NS + sid
        pltpu.make_async_copy(
            rows[_r_last % 2].at[pl.ds(0, ZR)],
            out_hbm.at[cid, pl.ds(j * ZR, ZR)], ssem[_r_last % 2]).wait()

    j14 = (_r_last - 1) * NS + sid
    pltpu.make_async_copy(
        rows[(_r_last - 1) % 2].at[pl.ds(0, ZR)],
        out_hbm.at[cid, pl.ds(j14 * ZR, ZR)],
        ssem[(_r_last - 1) % 2]).wait()


@functools.partial(
    pl.kernel,
    out_type=jax.ShapeDtypeStruct((NC * N,), jnp.float32),
    mesh=_mesh,
    scratch_types=[
        pltpu.VMEM((CD, KD), jnp.int32),
        pltpu.VMEM((KD,), jnp.float32),
        pltpu.VMEM((RPT,), jnp.float32),
        pltpu.VMEM_SHARED((N,), jnp.float32),
        pltpu.SemaphoreType.DMA,
    ],
)
def _sc_degree(dst_hbm, zeros_hbm, out_hbm, didx, ones_v, dbuf, acc, sem):
    cid = lax.axis_index("c")
    sid = lax.axis_index("s")
    w = cid * NS + sid

    @pl.when(sid < IO_T)
    def _():
        pltpu.sync_copy(zeros_hbm, dbuf)
        pltpu.sync_copy(dbuf, acc.at[pl.ds(sid * RPT, RPT)])

    pltpu.sync_copy(dst_hbm.at[w], didx)
    for i in range(KD // 16):
        ones_v[pl.ds(i * 16, 16)] = jnp.ones((16,), jnp.float32)
    plsc.subcore_barrier()

    def chunk(c, carry):
        pltpu.async_copy(ones_v, acc.at[didx.at[c]], sem, add=True)
        return carry

    lax.fori_loop(0, CD, chunk, 0)

    def drain(c, carry):
        pltpu.make_async_copy(ones_v, acc.at[didx.at[c]], sem).wait()
        return carry

    lax.fori_loop(0, CD, drain, 0)
    plsc.subcore_barrier()

    @pl.when(sid < IO_T)
    def _():
        pltpu.sync_copy(acc.at[pl.ds(sid * RPT, RPT)], dbuf)
        pltpu.sync_copy(dbuf, out_hbm.at[pl.ds(cid * N + sid * RPT, RPT)])


_BM = 2000


def _dinv_block(dref):
    d = dref[...]
    deg = d[:, 0:1] + d[:, 1:2] + 1.0
    return lax.rsqrt(jnp.maximum(deg, 1.0))


def _prep_body(xref, wref, dref, out):
    out[...] = jnp.dot(xref[...], wref[...],
                       preferred_element_type=jnp.float32) * _dinv_block(dref)


def _layer_body(aref, href, wref, bref, dref, out):
    dinv = _dinv_block(dref)
    p = aref[0] + aref[1] + href[...]
    t = jnp.maximum(p * dinv + bref[...], 0.0)
    out[...] = jnp.dot(t, wref[...],
                       preferred_element_type=jnp.float32) * dinv


def _final_body(aref, href, bref, dref, out):
    dinv = _dinv_block(dref)
    out[...] = (aref[0] + aref[1] + href[...]) * dinv + bref[...]


_row_spec = pl.BlockSpec((_BM, D), lambda i: (i, 0))
_agg_spec = pl.BlockSpec((NC, _BM, D), lambda i: (0, i, 0))
_w_spec = pl.BlockSpec((D, D), lambda i: (0, 0))
_b_spec = pl.BlockSpec((1, D), lambda i: (0, 0))
_deg_spec = pl.BlockSpec((_BM, NC), lambda i: (i, 0))
_out_f32 = jax.ShapeDtypeStruct((N, D), jnp.float32)

_prep = pl.pallas_call(
    _prep_body, grid=(N // _BM,),
    in_specs=[_row_spec, _w_spec, _deg_spec],
    out_specs=_row_spec, out_shape=_out_f32)

_layer = pl.pallas_call(
    _layer_body, grid=(N // _BM,),
    in_specs=[_agg_spec, _row_spec, _w_spec, _b_spec, _deg_spec],
    out_specs=_row_spec, out_shape=_out_f32)

_final = pl.pallas_call(
    _final_body, grid=(N // _BM,),
    in_specs=[_agg_spec, _row_spec, _b_spec, _deg_spec],
    out_specs=_row_spec, out_shape=_out_f32)


def kernel(x, edge_index, W1, b1, W2, b2, W3, b3):
    src = edge_index[0].reshape(NW, SEG, CS, K)
    dst = edge_index[1].reshape(NW, SEG, CS, K)
    dst_flat = edge_index[1].reshape(NW, CD, KD)
    zeros_rows = jnp.zeros((ZR, D), jnp.float32)
    zeros_deg = jnp.zeros((RPT,), jnp.float32)
    b1r = b1.reshape(1, D)
    b2r = b2.reshape(1, D)
    b3r = b3.reshape(1, D)

    degp = _sc_degree(dst_flat, zeros_deg).reshape(NC, N).T
    h1 = _prep(x, W1, degp)
    a1 = _sc_aggregate(h1, src, dst, zeros_rows)
    h2 = _layer(a1, h1, W2, b1r, degp)
    a2 = _sc_aggregate(h2, src, dst, zeros_rows)
    h3 = _layer(a2, h2, W3, b2r, degp)
    a3 = _sc_aggregate(h3, src, dst, zeros_rows)
    return _final(a3, h3, b3r, degp)
